# baseline (speedup 1.0000x reference)
import functools

import jax
import jax.numpy as jnp
from jax import lax
from jax.experimental import pallas as pl
from jax.experimental.pallas import tpu as pltpu
from jax.experimental.pallas import tpu_sc as plsc

_N = 10000
_E = 320000
_NFEAT = 128
_NHID = 128
_NCLASS = 40
_NLAYERS = 4
_TAU = 0.5

_NC = 2
_NS = 16
_NW = _NC * _NS
_CH = 112
_CF = 105
_CS = 75
_EPAD = _NS * (_CF + _CS) * _CH


def _spmm_sc(xh, meta, ev):
    mesh = plsc.VectorSubcoreMesh(
        core_axis_name="c", subcore_axis_name="s",
        num_cores=_NC, num_subcores=_NS)

    @functools.partial(
        pl.kernel,
        out_type=jax.ShapeDtypeStruct((_NC, _N, _NHID), jnp.float32),
        mesh=mesh,
        scratch_types=[
            pltpu.VMEM((_CH, _NHID), jnp.float32),
            pltpu.VMEM((_CH, _NHID), jnp.float32),
            pltpu.VMEM((_CH, _NHID), jnp.float32),
            pltpu.VMEM((2, _CH), jnp.int32),
            pltpu.VMEM((2, _CH), jnp.int32),
            pltpu.VMEM((2, _CH), jnp.int32),
            pltpu.VMEM((_CH // 16, 16), jnp.float32),
            pltpu.VMEM((_CH // 16, 16), jnp.float32),
            pltpu.VMEM((_CH // 16, 16), jnp.float32),
            pltpu.VMEM_SHARED((_N, _NHID), jnp.float32),
            pltpu.SemaphoreType.DMA,
            pltpu.SemaphoreType.DMA,
            pltpu.SemaphoreType.DMA,
            pltpu.SemaphoreType.DMA,
            pltpu.SemaphoreType.DMA,
            pltpu.SemaphoreType.DMA,
            pltpu.SemaphoreType.DMA,
            pltpu.SemaphoreType.DMA,
            pltpu.SemaphoreType.DMA,
        ],
    )
    def spmm(xh_hbm, meta_hbm, ev_hbm, out_hbm,
             rb0, rb1, rb2, m0, m1, m2, e0, e1, e2, acc_sh,
             sg0, sg1, sg2, ss0, ss1, ss2, sm0, sm1, sm2):
        bufs = (rb0, rb1, rb2)
        metas = (m0, m1, m2)
        evs = (e0, e1, e2)
        sgs = (sg0, sg1, sg2)
        sss = (ss0, ss1, ss2)
        sms = (sm0, sm1, sm2)
        cid = lax.axis_index("c")
        sid = lax.axis_index("s")
        nchunks = jnp.where(cid == 0, _CF, _CS)
        cbase = jnp.where(cid == 0, sid * _CF, _NS * _CF + sid * _CS)
        LAST = nchunks - 1

        zvec = jnp.zeros((16,), jnp.float32)

        def zero_body(i, carry):
            for k in range(_NHID // 16):
                rb0[i, pl.ds(k * 16, 16)] = zvec
            return carry

        lax.fori_loop(0, _CH, zero_body, 0)
        zbase = sid * 632
        for t in range(4):
            pltpu.sync_copy(rb0, acc_sh.at[pl.ds(zbase + t * _CH, _CH)])

        @pl.when(sid < 15)
        def _():
            pltpu.sync_copy(rb0, acc_sh.at[pl.ds(zbase + 448, _CH)])
            pltpu.sync_copy(rb0.at[pl.ds(0, 72)],
                            acc_sh.at[pl.ds(zbase + 560, 72)])

        @pl.when(sid == 15)
        def _():
            pltpu.sync_copy(rb0.at[pl.ds(0, 72)],
                            acc_sh.at[pl.ds(zbase + 448, 72)])

        plsc.subcore_barrier()

        def issue_meta(t, s):
            pltpu.async_copy(meta_hbm.at[cbase + t], metas[s], sms[s])
            pltpu.async_copy(ev_hbm.at[cbase + t], evs[s], sms[s])

        def drain_meta(s):
            pltpu.make_async_copy(
                meta_hbm.at[0], metas[s], sms[s]).wait()
            pltpu.make_async_copy(
                ev_hbm.at[0], evs[s], sms[s]).wait()

        H = _CH // 2

        def issue_gather(q):
            pltpu.async_copy(xh_hbm.at[metas[q].at[0, pl.ds(0, H)]],
                             bufs[q].at[pl.ds(0, H)], sgs[q])
            pltpu.async_copy(xh_hbm.at[metas[q].at[0, pl.ds(H, H)]],
                             bufs[q].at[pl.ds(H, H)], sgs[q])

        def drain_gather(q):
            pltpu.make_async_copy(
                xh_hbm.at[pl.ds(0, _CH)], bufs[q], sgs[q]).wait()

        def issue_scatter(q):
            pltpu.async_copy(bufs[q], acc_sh.at[metas[q].at[1]], sss[q],
                             add=True)

        def drain_scatter(q):
            pltpu.make_async_copy(
                xh_hbm.at[pl.ds(0, _CH)], bufs[q], sss[q]).wait()

        def scale(q):
            def scale_body(g, c2):
                ev16 = evs[q][g, :]
                for l in range(16):
                    evb = jnp.full((16,), ev16[l])
                    j = g * 16 + l
                    for k in range(_NHID // 16):
                        sl = pl.ds(k * 16, 16)
                        bufs[q][j, sl] = bufs[q][j, sl] * evb
                return c2

            lax.fori_loop(0, _CH // 16, scale_body, 0)

        for t in range(2):
            issue_meta(t, t)
            drain_meta(t)
            issue_gather(t)

        def pipe_body(p, carry):
            for q in range(3):
                t = 3 * p + q
                q2 = (q + 2) % 3

                @pl.when(t >= 1)
                def _():
                    drain_scatter(q2)

                @pl.when(t + 2 <= LAST)
                def _():
                    issue_meta(t + 2, q2)

                drain_gather(q)
                scale(q)
                issue_scatter(q)

                @pl.when(t + 2 <= LAST)
                def _():
                    drain_meta(q2)
                    issue_gather(q2)
            return carry

        lax.fori_loop(0, nchunks // 3, pipe_body, 0)
        drain_scatter(2)
        plsc.subcore_barrier()

        @pl.when(sid < 15)
        def _():
            pltpu.sync_copy(acc_sh.at[pl.ds(zbase, 632)],
                            out_hbm.at[cid, pl.ds(zbase, 632)])

        @pl.when(sid == 15)
        def _():
            pltpu.sync_copy(acc_sh.at[pl.ds(zbase, 520)],
                            out_hbm.at[cid, pl.ds(zbase, 520)])

    return spmm(xh, meta, ev)


_BLK = 1000
_GRID = _N // _BLK


def _row_spec():
    return pl.BlockSpec((_BLK, _NHID), lambda i: (i, 0))


def _full_spec(shape):
    return pl.BlockSpec(shape, lambda i: tuple(0 for _ in shape))


def _tc_init(x, w0t, b0):
    def body(x_ref, w_ref, b_ref, o_ref):
        o_ref[...] = (jnp.dot(x_ref[...], w_ref[...],
                              preferred_element_type=jnp.float32)
                      + b_ref[...])

    return pl.pallas_call(
        body,
        grid=(_GRID,),
        in_specs=[pl.BlockSpec((_BLK, _NFEAT), lambda i: (i, 0)),
                  _full_spec((_NFEAT, _NHID)),
                  _full_spec((1, _NHID))],
        out_specs=_row_spec(),
        out_shape=jax.ShapeDtypeStruct((_N, _NHID), jnp.float32),
    )(x, w0t, b0)


def _tc_xh(axp, xh, gamma):
    def body(ax0_ref, ax1_ref, xh_ref, g_ref, xhnew_ref):
        ax = ax0_ref[0] + ax1_ref[0]
        xhnew_ref[...] = g_ref[0, 0] * (xh_ref[...] - ax)

    return pl.pallas_call(
        body,
        grid=(_GRID,),
        in_specs=[pl.BlockSpec((1, _BLK, _NHID), lambda i: (0, i, 0)),
                  pl.BlockSpec((1, _BLK, _NHID), lambda i: (1, i, 0)),
                  _row_spec(),
                  pl.BlockSpec(memory_space=pltpu.SMEM)],
        out_specs=_row_spec(),
        out_shape=jax.ShapeDtypeStruct((_N, _NHID), jnp.float32),
    )(axp, axp, xh, gamma)


def _tc_out(axp, outp, w, beta):
    def body(ax0_ref, ax1_ref, out_ref, w_ref, onew_ref):
        ax = ax0_ref[0] + ax1_ref[0]
        mixed = jnp.dot(ax, w_ref[...], preferred_element_type=jnp.float32)
        onew_ref[...] = out_ref[...] + (1.0 - beta) * ax + beta * mixed

    return pl.pallas_call(
        body,
        grid=(_GRID,),
        in_specs=[pl.BlockSpec((1, _BLK, _NHID), lambda i: (0, i, 0)),
                  pl.BlockSpec((1, _BLK, _NHID), lambda i: (1, i, 0)),
                  _row_spec(),
                  _full_spec((_NHID, _NHID))],
        out_specs=_row_spec(),
        out_shape=jax.ShapeDtypeStruct((_N, _NHID), jnp.float32),
    )(axp, axp, outp, w)


def _tc_final(axp, outp, w, beta, wst, bpad):
    def body(ax0_ref, ax1_ref, out_ref, w_ref, ws_ref, b_ref, l_ref):
        ax = ax0_ref[0] + ax1_ref[0]
        mixed = jnp.dot(ax, w_ref[...], preferred_element_type=jnp.float32)
        o = out_ref[...] + (1.0 - beta) * ax + beta * mixed
        logits = (jnp.dot(o, ws_ref[...],
                          preferred_element_type=jnp.float32) + b_ref[...])
        m = jnp.max(logits, axis=1, keepdims=True)
        lse = jnp.log(jnp.sum(jnp.exp(logits - m), axis=1, keepdims=True))
        l_ref[...] = logits - m - lse

    return pl.pallas_call(
        body,
        grid=(_GRID,),
        in_specs=[pl.BlockSpec((1, _BLK, _NHID), lambda i: (0, i, 0)),
                  pl.BlockSpec((1, _BLK, _NHID), lambda i: (1, i, 0)),
                  _row_spec(),
                  _full_spec((_NHID, _NHID)),
                  _full_spec((_NHID, 128)),
                  _full_spec((1, 128))],
        out_specs=pl.BlockSpec((_BLK, 128), lambda i: (i, 0)),
        out_shape=jax.ShapeDtypeStruct((_N, 128), jnp.float32),
    )(axp, axp, outp, w, wst, bpad)


def kernel(X, edge_index, edge_values, W0, b0, gammas, Ws, Wsort, bsort):
    pad = _EPAD - _E
    src = jnp.concatenate([edge_index[0], jnp.zeros((pad,), jnp.int32)])
    dst = jnp.concatenate([edge_index[1], jnp.zeros((pad,), jnp.int32)])
    evp = jnp.concatenate([edge_values, jnp.zeros((pad,), jnp.float32)])
    meta = jnp.stack([src.reshape(-1, _CH), dst.reshape(-1, _CH)], axis=1)
    evr = evp.reshape(-1, _CH // 16, 16)

    w0t = W0.T
    b0r = b0.reshape(1, _NHID)
    wst = jnp.zeros((_NHID, 128), jnp.float32).at[:, :_NCLASS].set(Wsort.T)
    bpad = jnp.full((1, 128), -1e30, jnp.float32).at[0, :_NCLASS].set(bsort)

    xh = _tc_init(X, w0t, b0r)
    outp = xh
    for i in range(_NLAYERS - 1):
        beta = _TAU / (i + 1)
        axp = _spmm_sc(xh, meta, evr)
        xh = _tc_xh(axp, xh, gammas[i].reshape(1, 1))
        outp = _tc_out(axp, outp, Ws[i], beta)
    axp = _spmm_sc(xh, meta, evr)
    logp = _tc_final(axp, outp, Ws[_NLAYERS - 1], _TAU / _NLAYERS, wst, bpad)
    return logp[:, :_NCLASS]


# speedup vs baseline: 1.0555x; 1.0555x over previous
import functools

import jax
import jax.numpy as jnp
from jax import lax
from jax.experimental import pallas as pl
from jax.experimental.pallas import tpu as pltpu
from jax.experimental.pallas import tpu_sc as plsc

_N = 10000
_E = 320000
_NFEAT = 128
_NHID = 128
_NCLASS = 40
_NLAYERS = 4
_TAU = 0.5

_NC = 2
_NS = 16
_NW = _NC * _NS
_CH = 112
_CF = 120
_CS = 60
_EPAD = _NS * (_CF + _CS) * _CH


def _spmm_sc(xh, meta, ev):
    mesh = plsc.VectorSubcoreMesh(
        core_axis_name="c", subcore_axis_name="s",
        num_cores=_NC, num_subcores=_NS)

    @functools.partial(
        pl.kernel,
        out_type=jax.ShapeDtypeStruct((_NC, _N, _NHID), jnp.float32),
        mesh=mesh,
        scratch_types=[
            pltpu.VMEM((_CH, _NHID), jnp.float32),
            pltpu.VMEM((_CH, _NHID), jnp.float32),
            pltpu.VMEM((_CH, _NHID), jnp.float32),
            pltpu.VMEM((2, _CH), jnp.int32),
            pltpu.VMEM((2, _CH), jnp.int32),
            pltpu.VMEM((2, _CH), jnp.int32),
            pltpu.VMEM((_CH // 16, 16), jnp.float32),
            pltpu.VMEM((_CH // 16, 16), jnp.float32),
            pltpu.VMEM((_CH // 16, 16), jnp.float32),
            pltpu.VMEM_SHARED((_N, _NHID), jnp.float32),
            pltpu.SemaphoreType.DMA,
            pltpu.SemaphoreType.DMA,
            pltpu.SemaphoreType.DMA,
            pltpu.SemaphoreType.DMA,
            pltpu.SemaphoreType.DMA,
            pltpu.SemaphoreType.DMA,
            pltpu.SemaphoreType.DMA,
            pltpu.SemaphoreType.DMA,
            pltpu.SemaphoreType.DMA,
        ],
    )
    def spmm(xh_hbm, meta_hbm, ev_hbm, out_hbm,
             rb0, rb1, rb2, m0, m1, m2, e0, e1, e2, acc_sh,
             sg0, sg1, sg2, ss0, ss1, ss2, sm0, sm1, sm2):
        bufs = (rb0, rb1, rb2)
        metas = (m0, m1, m2)
        evs = (e0, e1, e2)
        sgs = (sg0, sg1, sg2)
        sss = (ss0, ss1, ss2)
        sms = (sm0, sm1, sm2)
        cid = lax.axis_index("c")
        sid = lax.axis_index("s")
        nchunks = jnp.where(cid == 0, _CF, _CS)
        cbase = jnp.where(cid == 0, sid * _CF, _NS * _CF + sid * _CS)
        LAST = nchunks - 1

        zvec = jnp.zeros((16,), jnp.float32)

        def zero_body(i, carry):
            for k in range(_NHID // 16):
                rb0[i, pl.ds(k * 16, 16)] = zvec
            return carry

        lax.fori_loop(0, _CH, zero_body, 0)
        zbase = sid * 632
        for t in range(4):
            pltpu.sync_copy(rb0, acc_sh.at[pl.ds(zbase + t * _CH, _CH)])

        @pl.when(sid < 15)
        def _():
            pltpu.sync_copy(rb0, acc_sh.at[pl.ds(zbase + 448, _CH)])
            pltpu.sync_copy(rb0.at[pl.ds(0, 72)],
                            acc_sh.at[pl.ds(zbase + 560, 72)])

        @pl.when(sid == 15)
        def _():
            pltpu.sync_copy(rb0.at[pl.ds(0, 72)],
                            acc_sh.at[pl.ds(zbase + 448, 72)])

        plsc.subcore_barrier()

        def issue_meta(t, s):
            pltpu.async_copy(meta_hbm.at[cbase + t], metas[s], sms[s])
            pltpu.async_copy(ev_hbm.at[cbase + t], evs[s], sms[s])

        def drain_meta(s):
            pltpu.make_async_copy(
                meta_hbm.at[0], metas[s], sms[s]).wait()
            pltpu.make_async_copy(
                ev_hbm.at[0], evs[s], sms[s]).wait()

        H = _CH // 2

        def issue_gather(q):
            pltpu.async_copy(xh_hbm.at[metas[q].at[0, pl.ds(0, H)]],
                             bufs[q].at[pl.ds(0, H)], sgs[q])
            pltpu.async_copy(xh_hbm.at[metas[q].at[0, pl.ds(H, H)]],
                             bufs[q].at[pl.ds(H, H)], sgs[q])

        def drain_gather(q):
            pltpu.make_async_copy(
                xh_hbm.at[pl.ds(0, _CH)], bufs[q], sgs[q]).wait()

        def issue_scatter(q):
            pltpu.async_copy(bufs[q], acc_sh.at[metas[q].at[1]], sss[q],
                             add=True)

        def drain_scatter(q):
            pltpu.make_async_copy(
                xh_hbm.at[pl.ds(0, _CH)], bufs[q], sss[q]).wait()

        def scale(q):
            def scale_body(g, c2):
                ev16 = evs[q][g, :]
                for l in range(16):
                    evb = jnp.full((16,), ev16[l])
                    j = g * 16 + l
                    for k in range(_NHID // 16):
                        sl = pl.ds(k * 16, 16)
                        bufs[q][j, sl] = bufs[q][j, sl] * evb
                return c2

            lax.fori_loop(0, _CH // 16, scale_body, 0)

        for t in range(2):
            issue_meta(t, t)
            drain_meta(t)
            issue_gather(t)

        def pipe_body(p, carry):
            for q in range(3):
                t = 3 * p + q
                q2 = (q + 2) % 3

                @pl.when(t >= 1)
                def _():
                    drain_scatter(q2)

                @pl.when(t + 2 <= LAST)
                def _():
                    issue_meta(t + 2, q2)

                drain_gather(q)
                scale(q)
                issue_scatter(q)

                @pl.when(t + 2 <= LAST)
                def _():
                    drain_meta(q2)
                    issue_gather(q2)
            return carry

        lax.fori_loop(0, nchunks // 3, pipe_body, 0)
        drain_scatter(2)
        plsc.subcore_barrier()

        @pl.when(sid < 15)
        def _():
            pltpu.sync_copy(acc_sh.at[pl.ds(zbase, 632)],
                            out_hbm.at[cid, pl.ds(zbase, 632)])

        @pl.when(sid == 15)
        def _():
            pltpu.sync_copy(acc_sh.at[pl.ds(zbase, 520)],
                            out_hbm.at[cid, pl.ds(zbase, 520)])

    return spmm(xh, meta, ev)


_BLK = 1000
_GRID = _N // _BLK


def _row_spec():
    return pl.BlockSpec((_BLK, _NHID), lambda i: (i, 0))


def _full_spec(shape):
    return pl.BlockSpec(shape, lambda i: tuple(0 for _ in shape))


def _tc_init(x, w0t, b0):
    def body(x_ref, w_ref, b_ref, o_ref):
        o_ref[...] = (jnp.dot(x_ref[...], w_ref[...],
                              preferred_element_type=jnp.float32)
                      + b_ref[...])

    return pl.pallas_call(
        body,
        grid=(_GRID,),
        in_specs=[pl.BlockSpec((_BLK, _NFEAT), lambda i: (i, 0)),
                  _full_spec((_NFEAT, _NHID)),
                  _full_spec((1, _NHID))],
        out_specs=_row_spec(),
        out_shape=jax.ShapeDtypeStruct((_N, _NHID), jnp.float32),
    )(x, w0t, b0)


def _tc_xh(axp, xh, gamma):
    def body(ax0_ref, ax1_ref, xh_ref, g_ref, xhnew_ref):
        ax = ax0_ref[0] + ax1_ref[0]
        xhnew_ref[...] = g_ref[0, 0] * (xh_ref[...] - ax)

    return pl.pallas_call(
        body,
        grid=(_GRID,),
        in_specs=[pl.BlockSpec((1, _BLK, _NHID), lambda i: (0, i, 0)),
                  pl.BlockSpec((1, _BLK, _NHID), lambda i: (1, i, 0)),
                  _row_spec(),
                  pl.BlockSpec(memory_space=pltpu.SMEM)],
        out_specs=_row_spec(),
        out_shape=jax.ShapeDtypeStruct((_N, _NHID), jnp.float32),
    )(axp, axp, xh, gamma)


def _tc_out(axp, outp, w, beta):
    def body(ax0_ref, ax1_ref, out_ref, w_ref, onew_ref):
        ax = ax0_ref[0] + ax1_ref[0]
        mixed = jnp.dot(ax, w_ref[...], preferred_element_type=jnp.float32)
        onew_ref[...] = out_ref[...] + (1.0 - beta) * ax + beta * mixed

    return pl.pallas_call(
        body,
        grid=(_GRID,),
        in_specs=[pl.BlockSpec((1, _BLK, _NHID), lambda i: (0, i, 0)),
                  pl.BlockSpec((1, _BLK, _NHID), lambda i: (1, i, 0)),
                  _row_spec(),
                  _full_spec((_NHID, _NHID))],
        out_specs=_row_spec(),
        out_shape=jax.ShapeDtypeStruct((_N, _NHID), jnp.float32),
    )(axp, axp, outp, w)


def _tc_final(axp, outp, w, beta, wst, bpad):
    def body(ax0_ref, ax1_ref, out_ref, w_ref, ws_ref, b_ref, l_ref):
        ax = ax0_ref[0] + ax1_ref[0]
        mixed = jnp.dot(ax, w_ref[...], preferred_element_type=jnp.float32)
        o = out_ref[...] + (1.0 - beta) * ax + beta * mixed
        logits = (jnp.dot(o, ws_ref[...],
                          preferred_element_type=jnp.float32) + b_ref[...])
        m = jnp.max(logits, axis=1, keepdims=True)
        lse = jnp.log(jnp.sum(jnp.exp(logits - m), axis=1, keepdims=True))
        l_ref[...] = logits - m - lse

    return pl.pallas_call(
        body,
        grid=(_GRID,),
        in_specs=[pl.BlockSpec((1, _BLK, _NHID), lambda i: (0, i, 0)),
                  pl.BlockSpec((1, _BLK, _NHID), lambda i: (1, i, 0)),
                  _row_spec(),
                  _full_spec((_NHID, _NHID)),
                  _full_spec((_NHID, 128)),
                  _full_spec((1, 128))],
        out_specs=pl.BlockSpec((_BLK, 128), lambda i: (i, 0)),
        out_shape=jax.ShapeDtypeStruct((_N, 128), jnp.float32),
    )(axp, axp, outp, w, wst, bpad)


def kernel(X, edge_index, edge_values, W0, b0, gammas, Ws, Wsort, bsort):
    pad = _EPAD - _E
    src = jnp.concatenate([edge_index[0], jnp.zeros((pad,), jnp.int32)])
    dst = jnp.concatenate([edge_index[1], jnp.zeros((pad,), jnp.int32)])
    evp = jnp.concatenate([edge_values, jnp.zeros((pad,), jnp.float32)])
    meta = jnp.stack([src.reshape(-1, _CH), dst.reshape(-1, _CH)], axis=1)
    evr = evp.reshape(-1, _CH // 16, 16)

    w0t = W0.T
    b0r = b0.reshape(1, _NHID)
    wst = jnp.zeros((_NHID, 128), jnp.float32).at[:, :_NCLASS].set(Wsort.T)
    bpad = jnp.full((1, 128), -1e30, jnp.float32).at[0, :_NCLASS].set(bsort)

    xh = _tc_init(X, w0t, b0r)
    outp = xh
    for i in range(_NLAYERS - 1):
        beta = _TAU / (i + 1)
        axp = _spmm_sc(xh, meta, evr)
        xh = _tc_xh(axp, xh, gammas[i].reshape(1, 1))
        outp = _tc_out(axp, outp, Ws[i], beta)
    axp = _spmm_sc(xh, meta, evr)
    logp = _tc_final(axp, outp, Ws[_NLAYERS - 1], _TAU / _NLAYERS, wst, bpad)
    return logp[:, :_NCLASS]


# speedup vs baseline: 1.0882x; 1.0310x over previous
import functools

import jax
import jax.numpy as jnp
from jax import lax
from jax.experimental import pallas as pl
from jax.experimental.pallas import tpu as pltpu
from jax.experimental.pallas import tpu_sc as plsc

_N = 10000
_E = 320000
_NFEAT = 128
_NHID = 128
_NCLASS = 40
_NLAYERS = 4
_TAU = 0.5

_NC = 2
_NS = 16
_NW = _NC * _NS
_CH = 112
_CF = 132
_CS = 48
_EPAD = _NS * (_CF + _CS) * _CH


def _spmm_sc(xh, meta, ev):
    mesh = plsc.VectorSubcoreMesh(
        core_axis_name="c", subcore_axis_name="s",
        num_cores=_NC, num_subcores=_NS)

    @functools.partial(
        pl.kernel,
        out_type=jax.ShapeDtypeStruct((_NC, _N, _NHID), jnp.float32),
        mesh=mesh,
        scratch_types=[
            pltpu.VMEM((_CH, _NHID), jnp.float32),
            pltpu.VMEM((_CH, _NHID), jnp.float32),
            pltpu.VMEM((_CH, _NHID), jnp.float32),
            pltpu.VMEM((2, _CH), jnp.int32),
            pltpu.VMEM((2, _CH), jnp.int32),
            pltpu.VMEM((2, _CH), jnp.int32),
            pltpu.VMEM((_CH // 16, 16), jnp.float32),
            pltpu.VMEM((_CH // 16, 16), jnp.float32),
            pltpu.VMEM((_CH // 16, 16), jnp.float32),
            pltpu.VMEM_SHARED((_N, _NHID), jnp.float32),
            pltpu.SemaphoreType.DMA,
            pltpu.SemaphoreType.DMA,
            pltpu.SemaphoreType.DMA,
            pltpu.SemaphoreType.DMA,
            pltpu.SemaphoreType.DMA,
            pltpu.SemaphoreType.DMA,
            pltpu.SemaphoreType.DMA,
            pltpu.SemaphoreType.DMA,
            pltpu.SemaphoreType.DMA,
        ],
    )
    def spmm(xh_hbm, meta_hbm, ev_hbm, out_hbm,
             rb0, rb1, rb2, m0, m1, m2, e0, e1, e2, acc_sh,
             sg0, sg1, sg2, ss0, ss1, ss2, sm0, sm1, sm2):
        bufs = (rb0, rb1, rb2)
        metas = (m0, m1, m2)
        evs = (e0, e1, e2)
        sgs = (sg0, sg1, sg2)
        sss = (ss0, ss1, ss2)
        sms = (sm0, sm1, sm2)
        cid = lax.axis_index("c")
        sid = lax.axis_index("s")
        nchunks = jnp.where(cid == 0, _CF, _CS)
        cbase = jnp.where(cid == 0, sid * _CF, _NS * _CF + sid * _CS)
        LAST = nchunks - 1

        zvec = jnp.zeros((16,), jnp.float32)

        def zero_body(i, carry):
            for k in range(_NHID // 16):
                rb0[i, pl.ds(k * 16, 16)] = zvec
            return carry

        lax.fori_loop(0, _CH, zero_body, 0)
        zbase = sid * 632
        for t in range(4):
            pltpu.sync_copy(rb0, acc_sh.at[pl.ds(zbase + t * _CH, _CH)])

        @pl.when(sid < 15)
        def _():
            pltpu.sync_copy(rb0, acc_sh.at[pl.ds(zbase + 448, _CH)])
            pltpu.sync_copy(rb0.at[pl.ds(0, 72)],
                            acc_sh.at[pl.ds(zbase + 560, 72)])

        @pl.when(sid == 15)
        def _():
            pltpu.sync_copy(rb0.at[pl.ds(0, 72)],
                            acc_sh.at[pl.ds(zbase + 448, 72)])

        plsc.subcore_barrier()

        def issue_meta(t, s):
            pltpu.async_copy(meta_hbm.at[cbase + t], metas[s], sms[s])
            pltpu.async_copy(ev_hbm.at[cbase + t], evs[s], sms[s])

        def drain_meta(s):
            pltpu.make_async_copy(
                meta_hbm.at[0], metas[s], sms[s]).wait()
            pltpu.make_async_copy(
                ev_hbm.at[0], evs[s], sms[s]).wait()

        H = _CH // 2

        def issue_gather(q):
            pltpu.async_copy(xh_hbm.at[metas[q].at[0, pl.ds(0, H)]],
                             bufs[q].at[pl.ds(0, H)], sgs[q])
            pltpu.async_copy(xh_hbm.at[metas[q].at[0, pl.ds(H, H)]],
                             bufs[q].at[pl.ds(H, H)], sgs[q])

        def drain_gather(q):
            pltpu.make_async_copy(
                xh_hbm.at[pl.ds(0, _CH)], bufs[q], sgs[q]).wait()

        def issue_scatter(q):
            pltpu.async_copy(bufs[q], acc_sh.at[metas[q].at[1]], sss[q],
                             add=True)

        def drain_scatter(q):
            pltpu.make_async_copy(
                xh_hbm.at[pl.ds(0, _CH)], bufs[q], sss[q]).wait()

        def scale(q):
            def scale_body(g, c2):
                ev16 = evs[q][g, :]
                for l in range(16):
                    evb = jnp.full((16,), ev16[l])
                    j = g * 16 + l
                    for k in range(_NHID // 16):
                        sl = pl.ds(k * 16, 16)
                        bufs[q][j, sl] = bufs[q][j, sl] * evb
                return c2

            lax.fori_loop(0, _CH // 16, scale_body, 0)

        for t in range(2):
            issue_meta(t, t)
            drain_meta(t)
            issue_gather(t)

        def pipe_body(p, carry):
            for q in range(3):
                t = 3 * p + q
                q2 = (q + 2) % 3

                @pl.when(t >= 1)
                def _():
                    drain_scatter(q2)

                @pl.when(t + 2 <= LAST)
                def _():
                    issue_meta(t + 2, q2)

                drain_gather(q)
                scale(q)
                issue_scatter(q)

                @pl.when(t + 2 <= LAST)
                def _():
                    drain_meta(q2)
                    issue_gather(q2)
            return carry

        lax.fori_loop(0, nchunks // 3, pipe_body, 0)
        drain_scatter(2)
        plsc.subcore_barrier()

        @pl.when(sid < 15)
        def _():
            pltpu.sync_copy(acc_sh.at[pl.ds(zbase, 632)],
                            out_hbm.at[cid, pl.ds(zbase, 632)])

        @pl.when(sid == 15)
        def _():
            pltpu.sync_copy(acc_sh.at[pl.ds(zbase, 520)],
                            out_hbm.at[cid, pl.ds(zbase, 520)])

    return spmm(xh, meta, ev)


_BLK = 1000
_GRID = _N // _BLK


def _row_spec():
    return pl.BlockSpec((_BLK, _NHID), lambda i: (i, 0))


def _full_spec(shape):
    return pl.BlockSpec(shape, lambda i: tuple(0 for _ in shape))


def _tc_init(x, w0t, b0):
    def body(x_ref, w_ref, b_ref, o_ref):
        o_ref[...] = (jnp.dot(x_ref[...], w_ref[...],
                              preferred_element_type=jnp.float32)
                      + b_ref[...])

    return pl.pallas_call(
        body,
        grid=(_GRID,),
        in_specs=[pl.BlockSpec((_BLK, _NFEAT), lambda i: (i, 0)),
                  _full_spec((_NFEAT, _NHID)),
                  _full_spec((1, _NHID))],
        out_specs=_row_spec(),
        out_shape=jax.ShapeDtypeStruct((_N, _NHID), jnp.float32),
    )(x, w0t, b0)


def _tc_xh(axp, xh, gamma):
    def body(ax0_ref, ax1_ref, xh_ref, g_ref, xhnew_ref):
        ax = ax0_ref[0] + ax1_ref[0]
        xhnew_ref[...] = g_ref[0, 0] * (xh_ref[...] - ax)

    return pl.pallas_call(
        body,
        grid=(_GRID,),
        in_specs=[pl.BlockSpec((1, _BLK, _NHID), lambda i: (0, i, 0)),
                  pl.BlockSpec((1, _BLK, _NHID), lambda i: (1, i, 0)),
                  _row_spec(),
                  pl.BlockSpec(memory_space=pltpu.SMEM)],
        out_specs=_row_spec(),
        out_shape=jax.ShapeDtypeStruct((_N, _NHID), jnp.float32),
    )(axp, axp, xh, gamma)


def _tc_out(axp, outp, w, beta):
    def body(ax0_ref, ax1_ref, out_ref, w_ref, onew_ref):
        ax = ax0_ref[0] + ax1_ref[0]
        mixed = jnp.dot(ax, w_ref[...], preferred_element_type=jnp.float32)
        onew_ref[...] = out_ref[...] + (1.0 - beta) * ax + beta * mixed

    return pl.pallas_call(
        body,
        grid=(_GRID,),
        in_specs=[pl.BlockSpec((1, _BLK, _NHID), lambda i: (0, i, 0)),
                  pl.BlockSpec((1, _BLK, _NHID), lambda i: (1, i, 0)),
                  _row_spec(),
                  _full_spec((_NHID, _NHID))],
        out_specs=_row_spec(),
        out_shape=jax.ShapeDtypeStruct((_N, _NHID), jnp.float32),
    )(axp, axp, outp, w)


def _tc_final(axp, outp, w, beta, wst, bpad):
    def body(ax0_ref, ax1_ref, out_ref, w_ref, ws_ref, b_ref, l_ref):
        ax = ax0_ref[0] + ax1_ref[0]
        mixed = jnp.dot(ax, w_ref[...], preferred_element_type=jnp.float32)
        o = out_ref[...] + (1.0 - beta) * ax + beta * mixed
        logits = (jnp.dot(o, ws_ref[...],
                          preferred_element_type=jnp.float32) + b_ref[...])
        m = jnp.max(logits, axis=1, keepdims=True)
        lse = jnp.log(jnp.sum(jnp.exp(logits - m), axis=1, keepdims=True))
        l_ref[...] = logits - m - lse

    return pl.pallas_call(
        body,
        grid=(_GRID,),
        in_specs=[pl.BlockSpec((1, _BLK, _NHID), lambda i: (0, i, 0)),
                  pl.BlockSpec((1, _BLK, _NHID), lambda i: (1, i, 0)),
                  _row_spec(),
                  _full_spec((_NHID, _NHID)),
                  _full_spec((_NHID, 128)),
                  _full_spec((1, 128))],
        out_specs=pl.BlockSpec((_BLK, 128), lambda i: (i, 0)),
        out_shape=jax.ShapeDtypeStruct((_N, 128), jnp.float32),
    )(axp, axp, outp, w, wst, bpad)


def kernel(X, edge_index, edge_values, W0, b0, gammas, Ws, Wsort, bsort):
    pad = _EPAD - _E
    src = jnp.concatenate([edge_index[0], jnp.zeros((pad,), jnp.int32)])
    dst = jnp.concatenate([edge_index[1], jnp.zeros((pad,), jnp.int32)])
    evp = jnp.concatenate([edge_values, jnp.zeros((pad,), jnp.float32)])
    meta = jnp.stack([src.reshape(-1, _CH), dst.reshape(-1, _CH)], axis=1)
    evr = evp.reshape(-1, _CH // 16, 16)

    w0t = W0.T
    b0r = b0.reshape(1, _NHID)
    wst = jnp.zeros((_NHID, 128), jnp.float32).at[:, :_NCLASS].set(Wsort.T)
    bpad = jnp.full((1, 128), -1e30, jnp.float32).at[0, :_NCLASS].set(bsort)

    xh = _tc_init(X, w0t, b0r)
    outp = xh
    for i in range(_NLAYERS - 1):
        beta = _TAU / (i + 1)
        axp = _spmm_sc(xh, meta, evr)
        xh = _tc_xh(axp, xh, gammas[i].reshape(1, 1))
        outp = _tc_out(axp, outp, Ws[i], beta)
    axp = _spmm_sc(xh, meta, evr)
    logp = _tc_final(axp, outp, Ws[_NLAYERS - 1], _TAU / _NLAYERS, wst, bpad)
    return logp[:, :_NCLASS]
